# baseline (speedup 1.0000x reference)
import jax
import jax.numpy as jnp
from jax.experimental import pallas as pl
from jax.experimental.pallas import tpu as pltpu


def _make_fused_kernel(H, W, TH):
    W2 = 2 * W
    NT = TH * W2
    RT = (TH + 2) * W

    def _kernel(x_ref, w_ref, b_ref, out_ref, xpad_ref):
        C = x_ref.shape[1]
        HW = x_ref.shape[0]
        i = pl.program_id(1)

        @pl.when(i == 0)
        def _():
            zr = jnp.zeros((W, C), jnp.bfloat16)
            xpad_ref[pl.ds(0, W), :] = zr
            xpad_ref[pl.ds(W, HW), :] = x_ref[...].astype(jnp.bfloat16)
            xpad_ref[pl.ds(W + HW, W), :] = zr

        xt = xpad_ref[pl.ds(i * TH * W, RT), :]

        xi = jax.lax.bitcast_convert_type(xt, jnp.int16).astype(jnp.int32)
        xi = xi & jnp.int32(0xFFFF)
        packed = xi | jax.lax.shift_left(xi, jnp.int32(16))
        xd = pltpu.bitcast(packed, jnp.bfloat16)

        xu = jnp.transpose(xd)
        zc = jnp.zeros((C, W2), jnp.bfloat16)
        xup = jnp.concatenate([zc, xu, zc], axis=-1)

        TSL = (TH + 2) * W2
        lane_b = jax.lax.broadcasted_iota(jnp.int32, (1, TSL), 1) % W2
        bases = []
        for kx in range(3):
            s = W2 + (kx - 1)
            base = jax.lax.slice(xup, (0, s), (C, s + TSL))
            if kx == 0:
                base = jnp.where(lane_b != 0, base, jnp.bfloat16(0))
            elif kx == 2:
                base = jnp.where(lane_b != (W2 - 1), base, jnp.bfloat16(0))
            bases.append(base)
        pieces = [jax.lax.slice(bases[kx], (0, ar * W2), (C, ar * W2 + NT))
                  for ar in range(3) for kx in range(3)]
        slab = jnp.concatenate(pieces, axis=0)

        bias = b_ref[...]
        accs = []
        for p in range(2):
            sub = jax.lax.slice(slab, (p * 3 * C, 0), ((p + 2) * 3 * C, NT))
            acc = jnp.dot(w_ref[p], sub,
                          preferred_element_type=jnp.float32) + bias
            accs.append(acc.reshape(C, TH, W2))
        z = jnp.concatenate(accs, axis=-1)
        out_ref[...] = z.reshape(C, 2 * TH, W2)

    return _kernel


def _repack_weights(w6, C):
    w6r = w6.reshape(2, 2, 3, C, 2, C)
    kx0 = w6r[:, :, 0, :, 0, :]
    kx2 = w6r[:, :, 2, :, 1, :]
    kx1 = w6r[:, :, 1, :, 1, :] - kx0
    wk = jnp.stack([kx0, kx1, kx2], axis=2)
    wk = wk.transpose(0, 4, 1, 2, 3).reshape(2, C, 6 * C)
    return wk.astype(jnp.bfloat16)


def kernel(x_nchw, w6, b2):
    B, C, H, W = x_nchw.shape
    TH = H // 4
    wt = _repack_weights(w6, C)
    bt = b2[0, :C].reshape(C, 1).astype(jnp.float32)
    x_cm = jnp.transpose(x_nchw, (0, 2, 3, 1)).reshape(B, H * W, C)

    return pl.pallas_call(
        _make_fused_kernel(H, W, TH),
        out_shape=jax.ShapeDtypeStruct((B, C, 2 * H, 2 * W), jnp.float32),
        grid=(B, H // TH),
        in_specs=[
            pl.BlockSpec((None, H * W, C), lambda b, i: (b, 0, 0)),
            pl.BlockSpec((2, C, 6 * C), lambda b, i: (0, 0, 0)),
            pl.BlockSpec((C, 1), lambda b, i: (0, 0)),
        ],
        out_specs=pl.BlockSpec((None, C, 2 * TH, 2 * W),
                               lambda b, i: (b, 0, i, 0)),
        scratch_shapes=[pltpu.VMEM(((H + 2) * W, C), jnp.bfloat16)],
        compiler_params=pltpu.CompilerParams(
            dimension_semantics=("parallel", "arbitrary"),
            vmem_limit_bytes=60 * 1024 * 1024,
        ),
    )(x_cm, wt, bt)


# speedup vs baseline: 1.0973x; 1.0973x over previous
import jax
import jax.numpy as jnp
from jax.experimental import pallas as pl
from jax.experimental.pallas import tpu as pltpu


def _make_fused_kernel(H, W):
    W2 = 2 * W
    NT = H * W2

    def _kernel(x_ref, w_ref, b_ref, out_ref):
        C = x_ref.shape[1]

        xt = x_ref[...].astype(jnp.bfloat16)
        xi = jax.lax.bitcast_convert_type(xt, jnp.int16).astype(jnp.int32)
        xi = xi & jnp.int32(0xFFFF)
        packed = xi | jax.lax.shift_left(xi, jnp.int32(16))
        xd = pltpu.bitcast(packed, jnp.bfloat16)

        xu = jnp.transpose(xd)
        zc = jnp.zeros((C, W2), jnp.bfloat16)
        xup = jnp.concatenate([zc, zc, xu, zc, zc], axis=-1)

        TSL = (H + 2) * W2
        lane_b = jax.lax.broadcasted_iota(jnp.int32, (1, TSL), 1) % W2
        bases = []
        for kx in range(3):
            s = W2 + (kx - 1)
            base = jax.lax.slice(xup, (0, s), (C, s + TSL))
            if kx == 0:
                base = jnp.where(lane_b != 0, base, jnp.bfloat16(0))
            elif kx == 2:
                base = jnp.where(lane_b != (W2 - 1), base, jnp.bfloat16(0))
            bases.append(base)
        pieces = [jax.lax.slice(bases[kx], (0, ar * W2), (C, ar * W2 + NT))
                  for ar in range(3) for kx in range(3)]
        slab = jnp.concatenate(pieces, axis=0)

        bias = b_ref[...]
        accs = []
        for p in range(2):
            sub = jax.lax.slice(slab, (p * 3 * C, 0), ((p + 2) * 3 * C, NT))
            acc = jnp.dot(w_ref[p], sub,
                          preferred_element_type=jnp.float32) + bias
            accs.append(acc.reshape(C, H, W2))
        nchunk = 4
        ch = H // nchunk
        for j in range(nchunk):
            z = jnp.concatenate([a[:, j * ch:(j + 1) * ch, :] for a in accs],
                                axis=-1)
            out_ref[:, 2 * j * ch:2 * (j + 1) * ch, :] = \
                z.reshape(C, 2 * ch, W2)

    return _kernel


def _repack_weights(w6, C):
    w6r = w6.reshape(2, 2, 3, C, 2, C)
    kx0 = w6r[:, :, 0, :, 0, :]
    kx2 = w6r[:, :, 2, :, 1, :]
    kx1 = w6r[:, :, 1, :, 1, :] - kx0
    wk = jnp.stack([kx0, kx1, kx2], axis=2)
    wk = wk.transpose(0, 4, 1, 2, 3).reshape(2, C, 6 * C)
    return wk.astype(jnp.bfloat16)


def kernel(x_nchw, w6, b2):
    B, C, H, W = x_nchw.shape
    wt = _repack_weights(w6, C)
    bt = b2[0, :C].reshape(C, 1).astype(jnp.float32)
    x_cm = jnp.transpose(x_nchw, (0, 2, 3, 1)).reshape(B, H * W, C)

    return pl.pallas_call(
        _make_fused_kernel(H, W),
        out_shape=jax.ShapeDtypeStruct((B, C, 2 * H, 2 * W), jnp.float32),
        grid=(B,),
        in_specs=[
            pl.BlockSpec((None, H * W, C), lambda b: (b, 0, 0)),
            pl.BlockSpec((2, C, 6 * C), lambda b: (0, 0, 0)),
            pl.BlockSpec((C, 1), lambda b: (0, 0)),
        ],
        out_specs=pl.BlockSpec((None, C, 2 * H, 2 * W),
                               lambda b: (b, 0, 0, 0)),
        compiler_params=pltpu.CompilerParams(
            dimension_semantics=("parallel",),
            vmem_limit_bytes=63 * 1024 * 1024,
        ),
    )(x_cm, wt, bt)
